# baseline (speedup 1.0000x reference)
import jax
import jax.numpy as jnp
from jax import lax
from jax.experimental import pallas as pl
from jax.experimental.pallas import tpu as pltpu
from jax.experimental.pallas import tpu_sc as plsc

D = 128
DW = D // 2
B = 16384
K = 20
G = K + 1
R = K + 2

NC = 2
NS = 16
NW = NC * NS
EPW = B // NW
C = 16
NCHUNK = EPW // C
CR = C * R
NSTREAM = [(0, 128), (128, 128), (256, 64)]
W = 17
GRP = 16 * W


def _chunk_copies(idxs, idxd, idxn, table, rall, sem, ch, make):
    mk = pltpu.make_async_copy if make else pltpu.async_copy
    cps = [
        mk(table.at[idxs.at[pl.ds(ch * C, C)]],
           rall.at[pl.ds(0, C)], sem),
        mk(table.at[idxd.at[pl.ds(ch * C, C)]],
           rall.at[pl.ds(C, C)], sem),
    ]
    for off, ln in NSTREAM:
        cps.append(mk(
            table.at[idxn.at[pl.ds(ch * C * K + off, ln)]],
            rall.at[pl.ds(2 * C + off, ln)], sem))
    return cps


def _fire_one(idxs, idxd, idxn, table, rall, sem, ch, si):
    if si == 0:
        pltpu.async_copy(table.at[idxs.at[pl.ds(ch * C, C)]],
                         rall.at[pl.ds(0, C)], sem)
    elif si == 1:
        pltpu.async_copy(table.at[idxd.at[pl.ds(ch * C, C)]],
                         rall.at[pl.ds(C, C)], sem)
    else:
        off, ln = NSTREAM[si - 2]
        pltpu.async_copy(table.at[idxn.at[pl.ds(ch * C * K + off, ln)]],
                         rall.at[pl.ds(2 * C + off, ln)], sem)


def _sc_body(table, src, dst, negf, aff_out, negaff_out,
             idxs, idxd, idxn, rall0, rall1, stage, avall, navall,
             sem0, sem1):
    cid = lax.axis_index("c")
    sid = lax.axis_index("s")
    wid = cid * NS + sid
    iota = lax.iota(jnp.int32, 16)
    iw = iota * W
    ik = iota * K
    ralls = (rall0, rall1)
    sems = (sem0, sem1)

    pltpu.sync_copy(src.at[pl.ds(wid * EPW, EPW)], idxs)
    pltpu.sync_copy(dst.at[pl.ds(wid * EPW, EPW)], idxd)
    pltpu.sync_copy(negf.at[pl.ds(wid * EPW * K, EPW * K)], idxn)
    _chunk_copies(idxs, idxd, idxn, table, rall0, sem0, 0, make=False)

    def step_body(s, carry):
        for b in range(2):
            ch = 2 * s + b
            nxt = ch + 1

            for cp in _chunk_copies(idxs, idxd, idxn, table, ralls[b],
                                    sems[b], ch, make=True):
                cp.wait()
            rall = ralls[b]

            def _row(r):
                return [rall[r, pl.ds(ci * 16, 16)] for ci in range(8)]

            def _dot(u, w):
                acc = u[0] * w[0]
                for ci in range(1, 8):
                    acc = acc + u[ci] * w[ci]
                return acc

            def edge_body(e, ecarry):
                u = _row(e)
                plsc.store_scatter(stage, [iw + e], _dot(u, _row(C + e)))
                for k in range(K):
                    r = 2 * C + e * K + k
                    plsc.store_scatter(stage, [iw + ((1 + k) * GRP + e)],
                                       _dot(u, _row(r)))
                return ecarry

            segs = [(0, 3), (3, 6), (6, 10), (10, 13), (13, 16)]
            for si, (lo, hi) in enumerate(segs):
                @pl.when(nxt < NCHUNK)
                def _(si=si):
                    _fire_one(idxs, idxd, idxn, table, ralls[1 - b],
                              sems[1 - b], nxt, si)

                lax.fori_loop(lo, hi, edge_body, 0)

            for g in range(G):
                t0 = [plsc.load_gather(stage, [iota + (g * GRP + c * W)])
                      for c in range(16)]
                t1 = [t0[2 * i] + t0[2 * i + 1] for i in range(8)]
                t2 = [t1[2 * i] + t1[2 * i + 1] for i in range(4)]
                t3 = [t2[2 * i] + t2[2 * i + 1] for i in range(2)]
                res = t3[0] + t3[1]
                if g == 0:
                    avall[pl.ds(ch * C, 16)] = res
                else:
                    flat = ik + (ch * C * K + g - 1)
                    plsc.store_scatter(
                        navall, [lax.shift_right_logical(flat, 7),
                                 lax.bitwise_and(flat, 127)], res)
        return carry

    lax.fori_loop(0, NCHUNK // 2, step_body, 0)
    pltpu.sync_copy(avall, aff_out.at[pl.ds(wid * EPW, EPW)])
    pltpu.sync_copy(navall,
                    negaff_out.at[pl.ds(wid * (EPW * K // 128),
                                        EPW * K // 128)])


_sc_call = pl.kernel(
    _sc_body,
    out_type=(jax.ShapeDtypeStruct((B,), jnp.float32),
              jax.ShapeDtypeStruct((B * K // 128, 128), jnp.float32)),
    mesh=plsc.VectorSubcoreMesh(core_axis_name="c", subcore_axis_name="s",
                                num_cores=NC, num_subcores=NS),
    compiler_params=pltpu.CompilerParams(needs_layout_passes=False),
    scratch_types=[
        pltpu.VMEM((EPW,), jnp.int32),
        pltpu.VMEM((EPW,), jnp.int32),
        pltpu.VMEM((EPW * K,), jnp.int32),
        pltpu.VMEM((CR, D), jnp.float32),
        pltpu.VMEM((CR, D), jnp.float32),
        pltpu.VMEM((G * GRP,), jnp.float32),
        pltpu.VMEM((EPW,), jnp.float32),
        pltpu.VMEM((EPW * K // 128, 128), jnp.float32),
        pltpu.SemaphoreType.DMA,
        pltpu.SemaphoreType.DMA,
    ],
)


def _log_sigmoid(x):
    return jnp.minimum(x, 0.0) - jnp.log1p(jnp.exp(-jnp.abs(x)))


def _tc_body(aff_ref, nav_ref, l_ref, l0_ref, l1_ref):
    l0 = -jnp.sum(_log_sigmoid(aff_ref[...]))
    l1 = -jnp.sum(_log_sigmoid(-nav_ref[...]))
    l_ref[0, 0] = (l0 + l1) / B
    l0_ref[0, 0] = l0
    l1_ref[0, 0] = l1


_tc_call = pl.pallas_call(
    _tc_body,
    out_shape=(jax.ShapeDtypeStruct((1, 1), jnp.float32),) * 3,
    out_specs=(pl.BlockSpec(memory_space=pltpu.SMEM),) * 3,
)


@jax.jit
def kernel(table, src, dst, neg):
    aff, negaff2d = _sc_call(table, src, dst, neg.reshape(B * K))
    l, l0, l1 = _tc_call(aff.reshape(B // 128, 128), negaff2d)
    return (l[0, 0], l0[0, 0], l1[0, 0])


# speedup vs baseline: 1.2347x; 1.2347x over previous
import jax
import jax.numpy as jnp
from jax import lax
from jax.experimental import pallas as pl
from jax.experimental.pallas import tpu as pltpu
from jax.experimental.pallas import tpu_sc as plsc

D = 128
DW = D // 2
B = 16384
K = 20
G = K + 1
R = K + 2

NC = 2
NS = 16
NW = NC * NS
EPW = B // NW
C = 16
NCHUNK = EPW // C
CR = C * R
NSTREAM = [(0, 128), (128, 128), (256, 64)]
W = 17
GRP = 16 * W


def _chunk_copies(idxs, idxd, idxn, table, rall, sem, ch, make):
    mk = pltpu.make_async_copy if make else pltpu.async_copy
    cps = [
        mk(table.at[idxs.at[pl.ds(ch * C, C)]],
           rall.at[pl.ds(0, C)], sem),
        mk(table.at[idxd.at[pl.ds(ch * C, C)]],
           rall.at[pl.ds(C, C)], sem),
    ]
    for off, ln in NSTREAM:
        cps.append(mk(
            table.at[idxn.at[pl.ds(ch * C * K + off, ln)]],
            rall.at[pl.ds(2 * C + off, ln)], sem))
    return cps


def _sc_body(table, src, dst, negf, aff_out, negaff_out,
             idxs, idxd, idxn, rall0, rall1, stage, avall, navall,
             sem0, sem1):
    cid = lax.axis_index("c")
    sid = lax.axis_index("s")
    wid = cid * NS + sid
    iota = lax.iota(jnp.int32, 16)
    iw = iota * W
    ik = iota * K
    ralls = (rall0, rall1)
    sems = (sem0, sem1)

    pltpu.sync_copy(src.at[pl.ds(wid * EPW, EPW)], idxs)
    pltpu.sync_copy(dst.at[pl.ds(wid * EPW, EPW)], idxd)
    pltpu.sync_copy(negf.at[pl.ds(wid * EPW * K, EPW * K)], idxn)
    _chunk_copies(idxs, idxd, idxn, table, rall0, sem0, 0, make=False)

    def step_body(s, carry):
        for b in range(2):
            ch = 2 * s + b
            nxt = ch + 1

            @pl.when(nxt < NCHUNK)
            def _():
                _chunk_copies(idxs, idxd, idxn, table, ralls[1 - b],
                              sems[1 - b], nxt, make=False)

            for cp in _chunk_copies(idxs, idxd, idxn, table, ralls[b],
                                    sems[b], ch, make=True):
                cp.wait()
            rall = ralls[b]

            def _row(r):
                return [rall[r, pl.ds(ci * 16, 16)] for ci in range(8)]

            def _dot(u, w):
                acc = u[0] * w[0]
                for ci in range(1, 8):
                    acc = acc + u[ci] * w[ci]
                return acc

            def edge_body(e, ecarry):
                u = _row(e)
                plsc.store_scatter(stage, [iw + e], _dot(u, _row(C + e)))
                for k in range(K):
                    r = 2 * C + e * K + k
                    plsc.store_scatter(stage, [iw + ((1 + k) * GRP + e)],
                                       _dot(u, _row(r)))
                return ecarry

            lax.fori_loop(0, C, edge_body, 0)

            for g in range(G):
                t0 = [plsc.load_gather(stage, [iota + (g * GRP + c * W)])
                      for c in range(16)]
                t1 = [t0[2 * i] + t0[2 * i + 1] for i in range(8)]
                t2 = [t1[2 * i] + t1[2 * i + 1] for i in range(4)]
                t3 = [t2[2 * i] + t2[2 * i + 1] for i in range(2)]
                res = t3[0] + t3[1]
                if g == 0:
                    avall[pl.ds(ch * C, 16)] = res
                else:
                    flat = ik + (ch * C * K + g - 1)
                    plsc.store_scatter(
                        navall, [lax.shift_right_logical(flat, 7),
                                 lax.bitwise_and(flat, 127)], res)
        return carry

    lax.fori_loop(0, NCHUNK // 2, step_body, 0)
    pltpu.sync_copy(avall, aff_out.at[pl.ds(wid * EPW, EPW)])
    pltpu.sync_copy(navall,
                    negaff_out.at[pl.ds(wid * (EPW * K // 128),
                                        EPW * K // 128)])


_sc_call = pl.kernel(
    _sc_body,
    out_type=(jax.ShapeDtypeStruct((B,), jnp.float32),
              jax.ShapeDtypeStruct((B * K // 128, 128), jnp.float32)),
    mesh=plsc.VectorSubcoreMesh(core_axis_name="c", subcore_axis_name="s",
                                num_cores=NC, num_subcores=NS),
    compiler_params=pltpu.CompilerParams(needs_layout_passes=False),
    scratch_types=[
        pltpu.VMEM((EPW,), jnp.int32),
        pltpu.VMEM((EPW,), jnp.int32),
        pltpu.VMEM((EPW * K,), jnp.int32),
        pltpu.VMEM((CR, D), jnp.float32),
        pltpu.VMEM((CR, D), jnp.float32),
        pltpu.VMEM((G * GRP,), jnp.float32),
        pltpu.VMEM((EPW,), jnp.float32),
        pltpu.VMEM((EPW * K // 128, 128), jnp.float32),
        pltpu.SemaphoreType.DMA,
        pltpu.SemaphoreType.DMA,
    ],
)


def _log_sigmoid(x):
    return jnp.minimum(x, 0.0) - jnp.log1p(jnp.exp(-jnp.abs(x)))


def _tc_body(aff_ref, nav_ref, l_ref, l0_ref, l1_ref):
    l0 = -jnp.sum(_log_sigmoid(aff_ref[...]))
    l1 = -jnp.sum(_log_sigmoid(-nav_ref[...]))
    l_ref[0, 0] = (l0 + l1) / B
    l0_ref[0, 0] = l0
    l1_ref[0, 0] = l1


_tc_call = pl.pallas_call(
    _tc_body,
    out_shape=(jax.ShapeDtypeStruct((1, 1), jnp.float32),) * 3,
    out_specs=(pl.BlockSpec(memory_space=pltpu.SMEM),) * 3,
)


@jax.jit
def kernel(table, src, dst, neg):
    aff, negaff2d = _sc_call(table, src, dst, neg.reshape(B * K))
    l, l0, l1 = _tc_call(aff.reshape(B // 128, 128), negaff2d)
    return (l[0, 0], l0[0, 0], l1[0, 0])
